# baseline (speedup 1.0000x reference)
import functools

import jax
import jax.numpy as jnp
from jax import lax
from jax.experimental import pallas as pl
from jax.experimental.pallas import tpu as pltpu
from jax.experimental.pallas import tpu_sc as plsc

H = 128
_NC = 2
_NS = 16
_NW = _NC * _NS


def _uv_body(x_ref, W1_ref, b1_ref, u_ref, v_ref):
    x = x_ref[...]
    u_ref[...] = (jnp.dot(x, W1_ref[0:H, :], preferred_element_type=jnp.float32)
                  + b1_ref[...])
    v_ref[...] = jnp.dot(x, W1_ref[H:2 * H, :], preferred_element_type=jnp.float32)


def _edge_mlp_body(ge_ref, ea_ref, W1c_ref, W2_ref, b2_ref,
                   W3_ref, b3_ref, g_ref, beta_ref, enew_ref, eout_ref):
    ea = ea_ref[...]
    z = ge_ref[...] + jnp.dot(ea, W1c_ref[...], preferred_element_type=jnp.float32)
    z = jnp.maximum(z, 0.0)
    z = jnp.maximum(jnp.dot(z, W2_ref[...], preferred_element_type=jnp.float32)
                    + b2_ref[...], 0.0)
    z = jnp.dot(z, W3_ref[...], preferred_element_type=jnp.float32) + b3_ref[...]
    mu = jnp.mean(z, axis=-1, keepdims=True)
    zc = z - mu
    var = jnp.mean(zc * zc, axis=-1, keepdims=True)
    e_new = g_ref[...] * zc * jax.lax.rsqrt(var + 1e-5) + beta_ref[...]
    enew_ref[...] = e_new
    eout_ref[...] = ea + e_new


def _node_mlp_body(x_ref, agg0_ref, agg1_ref, W1_ref, b1_ref, W2_ref, b2_ref,
                   W3_ref, b3_ref, g_ref, beta_ref, xout_ref):
    x = x_ref[...]
    agg = agg0_ref[...] + agg1_ref[...]
    W1 = W1_ref[...]
    z = (jnp.dot(x, W1[0:H, :], preferred_element_type=jnp.float32)
         + jnp.dot(agg, W1[H:2 * H, :], preferred_element_type=jnp.float32)
         + b1_ref[...])
    z = jnp.maximum(z, 0.0)
    z = jnp.maximum(jnp.dot(z, W2_ref[...], preferred_element_type=jnp.float32)
                    + b2_ref[...], 0.0)
    z = jnp.dot(z, W3_ref[...], preferred_element_type=jnp.float32) + b3_ref[...]
    mu = jnp.mean(z, axis=-1, keepdims=True)
    zc = z - mu
    var = jnp.mean(zc * zc, axis=-1, keepdims=True)
    x_new = g_ref[...] * zc * jax.lax.rsqrt(var + 1e-5) + beta_ref[...]
    xout_ref[...] = x + x_new


def _row_spec(block_rows, cols):
    return pl.BlockSpec((block_rows, cols), lambda i: (i, 0))


def _const_spec(shape):
    return pl.BlockSpec(shape, lambda i: tuple(0 for _ in shape))


def _uv_precompute(x, W1, b1, block_rows=2000):
    Nn = x.shape[0]
    return pl.pallas_call(
        _uv_body, grid=(Nn // block_rows,),
        in_specs=[_row_spec(block_rows, H), _const_spec((3 * H, H)),
                  _const_spec((1, H))],
        out_specs=[_row_spec(block_rows, H), _row_spec(block_rows, H)],
        out_shape=[jax.ShapeDtypeStruct((Nn, H), jnp.float32),
                   jax.ShapeDtypeStruct((Nn, H), jnp.float32)],
    )(x, W1, b1.reshape(1, H))


def _edge_mlp(ge, ea, W1c, b1, W2, b2, W3, b3, g, beta, block_rows=2560):
    E = ge.shape[0]
    in_specs = [
        _row_spec(block_rows, H), _row_spec(block_rows, H),
        _const_spec((H, H)),
        _const_spec((H, H)), _const_spec((1, H)),
        _const_spec((H, H)), _const_spec((1, H)),
        _const_spec((1, H)), _const_spec((1, H)),
    ]
    out_specs = [_row_spec(block_rows, H), _row_spec(block_rows, H)]
    out_shape = [jax.ShapeDtypeStruct((E, H), jnp.float32),
                 jax.ShapeDtypeStruct((E, H), jnp.float32)]
    return pl.pallas_call(
        _edge_mlp_body, grid=(E // block_rows,), in_specs=in_specs,
        out_specs=out_specs, out_shape=out_shape,
    )(ge, ea, W1c, W2, b2.reshape(1, H),
      W3, b3.reshape(1, H), g.reshape(1, H), beta.reshape(1, H))


def _node_mlp(x, agg2, W1, b1, W2, b2, W3, b3, g, beta, block_rows=2000):
    Nn = x.shape[0]
    nblk = Nn // block_rows
    in_specs = [
        _row_spec(block_rows, H),
        pl.BlockSpec((block_rows, H), lambda i: (i, 0)),
        pl.BlockSpec((block_rows, H), lambda i: (i + nblk, 0)),
        _const_spec((2 * H, H)), _const_spec((1, H)),
        _const_spec((H, H)), _const_spec((1, H)),
        _const_spec((H, H)), _const_spec((1, H)),
        _const_spec((1, H)), _const_spec((1, H)),
    ]
    return pl.pallas_call(
        _node_mlp_body, grid=(nblk,), in_specs=in_specs,
        out_specs=_row_spec(block_rows, H),
        out_shape=jax.ShapeDtypeStruct((Nn, H), jnp.float32),
    )(x, agg2, agg2, W1, b1.reshape(1, H), W2, b2.reshape(1, H),
      W3, b3.reshape(1, H), g.reshape(1, H), beta.reshape(1, H))


def _sc_gather(u, v, src, dst, chunk=200):
    E = src.shape[0]
    rows_per_w = E // _NW
    n_chunks = rows_per_w // chunk
    mesh = plsc.VectorSubcoreMesh(core_axis_name="c", subcore_axis_name="s")

    @functools.partial(
        pl.kernel,
        out_type=jax.ShapeDtypeStruct((E, H), jnp.float32),
        mesh=mesh,
        scratch_types=[
            pltpu.VMEM((rows_per_w,), jnp.int32),
            pltpu.VMEM((rows_per_w,), jnp.int32),
            pltpu.VMEM((chunk, H), jnp.float32),
            pltpu.VMEM((chunk, H), jnp.float32),
            pltpu.VMEM((chunk, H), jnp.float32),
            pltpu.VMEM((chunk, H), jnp.float32),
            pltpu.SemaphoreType.DMA,
            pltpu.SemaphoreType.DMA,
            pltpu.SemaphoreType.DMA,
            pltpu.SemaphoreType.DMA,
        ],
    )
    def k(u_hbm, v_hbm, src_hbm, dst_hbm, out_hbm,
          isrc, idst, ru0, ru1, rv0, rv1, sg0, sg1, so0, so1):
        wid = lax.axis_index("s") * _NC + lax.axis_index("c")
        base0 = wid * rows_per_w
        ru = (ru0, ru1)
        rv = (rv0, rv1)
        sg = (sg0, sg1)
        so = (so0, so1)

        pltpu.sync_copy(src_hbm.at[pl.ds(base0, rows_per_w)], isrc)
        pltpu.sync_copy(dst_hbm.at[pl.ds(base0, rows_per_w)], idst)

        def issue(i, b):
            pltpu.async_copy(u_hbm.at[isrc.at[pl.ds(i * chunk, chunk)]],
                             ru[b], sg[b])
            pltpu.async_copy(v_hbm.at[idst.at[pl.ds(i * chunk, chunk)]],
                             rv[b], sg[b])

        def wait_gather(b):
            pltpu.make_async_copy(u_hbm.at[isrc.at[pl.ds(0, chunk)]],
                                  ru[b], sg[b]).wait()
            pltpu.make_async_copy(v_hbm.at[idst.at[pl.ds(0, chunk)]],
                                  rv[b], sg[b]).wait()

        def wait_out(b):
            pltpu.make_async_copy(ru[b], out_hbm.at[pl.ds(base0, chunk)],
                                  so[b]).wait()

        issue(0, 0)

        def outer(io, carry):
            for b in range(2):
                i = 2 * io + b
                nb = 1 - b

                @pl.when(i + 1 < n_chunks)
                def _issue_next():
                    @pl.when(i >= 1)
                    def _drain():
                        wait_out(nb)

                    issue(i + 1, nb)

                wait_gather(b)

                def add_row(r, c2):
                    for kk in range(H // 16):
                        sl = pl.ds(kk * 16, 16)
                        ru[b][r, sl] = ru[b][r, sl] + rv[b][r, sl]
                    return c2

                lax.fori_loop(0, chunk, add_row, 0)
                pltpu.async_copy(ru[b],
                                 out_hbm.at[pl.ds(base0 + i * chunk, chunk)],
                                 so[b])
            return carry

        lax.fori_loop(0, n_chunks // 2, outer, 0)
        wait_out(0)
        wait_out(1)

    return k(u, v, src, dst)


def _sc_scatter(e_new, dst, zeros, n_nodes, chunk=80):
    E = e_new.shape[0]
    rows_per_w = E // _NW
    n_chunks = rows_per_w // chunk
    rows_per_tile = 640
    n_pad = _NS * rows_per_tile
    last_rows = n_nodes - (_NS - 1) * rows_per_tile
    mesh = plsc.VectorSubcoreMesh(core_axis_name="c", subcore_axis_name="s")

    @functools.partial(
        pl.kernel,
        out_type=jax.ShapeDtypeStruct((2 * n_nodes, H), jnp.float32),
        mesh=mesh,
        scratch_types=[
            pltpu.VMEM((chunk,), jnp.int32),
            pltpu.VMEM((chunk,), jnp.int32),
            pltpu.VMEM((chunk, H), jnp.float32),
            pltpu.VMEM((chunk, H), jnp.float32),
            pltpu.VMEM_SHARED((n_pad, H), jnp.float32),
            pltpu.SemaphoreType.DMA,
            pltpu.SemaphoreType.DMA,
            pltpu.SemaphoreType.DMA,
            pltpu.SemaphoreType.DMA,
        ],
    )
    def k(enew_hbm, dst_hbm, zeros_hbm, out_hbm, idx0, idx1, buf0, buf1, acc,
          sr0, sr1, ss0, ss1):
        cid = lax.axis_index("c")
        sid = lax.axis_index("s")
        wid = sid * _NC + cid
        base0 = wid * rows_per_w
        nbase = sid * rows_per_tile
        idx = (idx0, idx1)
        buf = (buf0, buf1)
        sr = (sr0, sr1)
        ss = (ss0, ss1)
        pltpu.sync_copy(zeros_hbm, acc.at[pl.ds(nbase, rows_per_tile)])
        plsc.subcore_barrier()

        def issue(i, b):
            base = base0 + i * chunk
            pltpu.sync_copy(dst_hbm.at[pl.ds(base, chunk)], idx[b])
            pltpu.async_copy(enew_hbm.at[pl.ds(base, chunk)], buf[b], sr[b])

        def wait_rows(b):
            pltpu.make_async_copy(enew_hbm.at[pl.ds(base0, chunk)], buf[b],
                                  sr[b]).wait()

        def wait_scat(b):
            pltpu.make_async_copy(buf[b], acc.at[idx[b]], ss[b]).wait()

        issue(0, 0)

        def outer(io, carry):
            for b in range(2):
                i = 2 * io + b
                wait_rows(b)
                pltpu.async_copy(buf[b], acc.at[idx[b]], ss[b], add=True)

                @pl.when(i + 1 < n_chunks)
                def _issue_next():
                    @pl.when(i >= 1)
                    def _drain():
                        wait_scat(1 - b)

                    issue(i + 1, 1 - b)

            return carry

        lax.fori_loop(0, n_chunks // 2, outer, 0)
        if n_chunks % 2 == 1:
            wait_scat(1)
            wait_rows(0)
            pltpu.async_copy(buf[0], acc.at[idx[0]], ss[0], add=True)
            wait_scat(0)
        else:
            wait_scat(0)
            wait_scat(1)
        plsc.subcore_barrier()

        @pl.when(sid < _NS - 1)
        def _full():
            pltpu.sync_copy(acc.at[pl.ds(nbase, rows_per_tile)],
                            out_hbm.at[pl.ds(cid * n_nodes + nbase,
                                             rows_per_tile)])

        @pl.when(sid == _NS - 1)
        def _tail():
            pltpu.sync_copy(acc.at[pl.ds((_NS - 1) * rows_per_tile, last_rows)],
                            out_hbm.at[pl.ds(cid * n_nodes
                                             + (_NS - 1) * rows_per_tile,
                                             last_rows)])

    return k(e_new, dst, zeros)


def kernel(x, edge_attr, edge_index, eb_W1, eb_b1, eb_W2, eb_b2, eb_W3, eb_b3,
           eb_g, eb_beta, nb_W1, nb_b1, nb_W2, nb_b2, nb_W3, nb_b3, nb_g, nb_beta):
    src = edge_index[0]
    dst = edge_index[1]
    u, v = _uv_precompute(x, eb_W1, eb_b1)
    ge = _sc_gather(u, v, src, dst)
    e_new, e_out = _edge_mlp(ge, edge_attr, eb_W1[2 * H:3 * H], eb_b1,
                             eb_W2, eb_b2, eb_W3, eb_b3, eb_g, eb_beta)
    zeros = jnp.zeros((640, H), jnp.float32)
    agg2 = _sc_scatter(e_new, dst, zeros, x.shape[0])
    x_out = _node_mlp(x, agg2, nb_W1, nb_b1, nb_W2, nb_b2, nb_W3, nb_b3,
                      nb_g, nb_beta)
    return (x_out, e_out)


# speedup vs baseline: 1.0946x; 1.0946x over previous
import functools

import jax
import jax.numpy as jnp
from jax import lax
from jax.experimental import pallas as pl
from jax.experimental.pallas import tpu as pltpu
from jax.experimental.pallas import tpu_sc as plsc

H = 128
_NC = 2
_NS = 16
_NW = _NC * _NS


def _uv_body(x_ref, W1_ref, b1_ref, u_ref, v_ref):
    x = x_ref[...]
    u_ref[...] = (jnp.dot(x, W1_ref[0:H, :], preferred_element_type=jnp.float32)
                  + b1_ref[...])
    v_ref[...] = jnp.dot(x, W1_ref[H:2 * H, :], preferred_element_type=jnp.float32)


def _edge_mlp_body(ge_ref, ea_ref, W1c_ref, W2_ref, b2_ref,
                   W3_ref, b3_ref, g_ref, beta_ref, enew_ref, eout_ref):
    ea = ea_ref[...]
    z = ge_ref[...] + jnp.dot(ea, W1c_ref[...], preferred_element_type=jnp.float32)
    z = jnp.maximum(z, 0.0)
    z = jnp.maximum(jnp.dot(z, W2_ref[...], preferred_element_type=jnp.float32)
                    + b2_ref[...], 0.0)
    z = jnp.dot(z, W3_ref[...], preferred_element_type=jnp.float32) + b3_ref[...]
    mu = jnp.mean(z, axis=-1, keepdims=True)
    zc = z - mu
    var = jnp.mean(zc * zc, axis=-1, keepdims=True)
    e_new = g_ref[...] * zc * jax.lax.rsqrt(var + 1e-5) + beta_ref[...]
    enew_ref[...] = e_new
    eout_ref[...] = ea + e_new


def _node_mlp_body(x_ref, agg0_ref, agg1_ref, W1_ref, b1_ref, W2_ref, b2_ref,
                   W3_ref, b3_ref, g_ref, beta_ref, xout_ref):
    x = x_ref[...]
    agg = agg0_ref[...] + agg1_ref[...]
    W1 = W1_ref[...]
    z = (jnp.dot(x, W1[0:H, :], preferred_element_type=jnp.float32)
         + jnp.dot(agg, W1[H:2 * H, :], preferred_element_type=jnp.float32)
         + b1_ref[...])
    z = jnp.maximum(z, 0.0)
    z = jnp.maximum(jnp.dot(z, W2_ref[...], preferred_element_type=jnp.float32)
                    + b2_ref[...], 0.0)
    z = jnp.dot(z, W3_ref[...], preferred_element_type=jnp.float32) + b3_ref[...]
    mu = jnp.mean(z, axis=-1, keepdims=True)
    zc = z - mu
    var = jnp.mean(zc * zc, axis=-1, keepdims=True)
    x_new = g_ref[...] * zc * jax.lax.rsqrt(var + 1e-5) + beta_ref[...]
    xout_ref[...] = x + x_new


def _row_spec(block_rows, cols):
    return pl.BlockSpec((block_rows, cols), lambda i: (i, 0))


def _const_spec(shape):
    return pl.BlockSpec(shape, lambda i: tuple(0 for _ in shape))


def _uv_precompute(x, W1, b1, block_rows=2000):
    Nn = x.shape[0]
    return pl.pallas_call(
        _uv_body, grid=(Nn // block_rows,),
        in_specs=[_row_spec(block_rows, H), _const_spec((3 * H, H)),
                  _const_spec((1, H))],
        out_specs=[_row_spec(block_rows, H), _row_spec(block_rows, H)],
        out_shape=[jax.ShapeDtypeStruct((Nn, H), jnp.float32),
                   jax.ShapeDtypeStruct((Nn, H), jnp.float32)],
    )(x, W1, b1.reshape(1, H))


def _edge_mlp(ge, ea, W1c, b1, W2, b2, W3, b3, g, beta, block_rows=2560):
    E = ge.shape[0]
    in_specs = [
        _row_spec(block_rows, H), _row_spec(block_rows, H),
        _const_spec((H, H)),
        _const_spec((H, H)), _const_spec((1, H)),
        _const_spec((H, H)), _const_spec((1, H)),
        _const_spec((1, H)), _const_spec((1, H)),
    ]
    out_specs = [_row_spec(block_rows, H), _row_spec(block_rows, H)]
    out_shape = [jax.ShapeDtypeStruct((E, H), jnp.float32),
                 jax.ShapeDtypeStruct((E, H), jnp.float32)]
    return pl.pallas_call(
        _edge_mlp_body, grid=(E // block_rows,), in_specs=in_specs,
        out_specs=out_specs, out_shape=out_shape,
    )(ge, ea, W1c, W2, b2.reshape(1, H),
      W3, b3.reshape(1, H), g.reshape(1, H), beta.reshape(1, H))


def _node_mlp(x, agg2, W1, b1, W2, b2, W3, b3, g, beta, block_rows=2000):
    Nn = x.shape[0]
    nblk = Nn // block_rows
    in_specs = [
        _row_spec(block_rows, H),
        pl.BlockSpec((block_rows, H), lambda i: (i, 0)),
        pl.BlockSpec((block_rows, H), lambda i: (i + nblk, 0)),
        _const_spec((2 * H, H)), _const_spec((1, H)),
        _const_spec((H, H)), _const_spec((1, H)),
        _const_spec((H, H)), _const_spec((1, H)),
        _const_spec((1, H)), _const_spec((1, H)),
    ]
    return pl.pallas_call(
        _node_mlp_body, grid=(nblk,), in_specs=in_specs,
        out_specs=_row_spec(block_rows, H),
        out_shape=jax.ShapeDtypeStruct((Nn, H), jnp.float32),
    )(x, agg2, agg2, W1, b1.reshape(1, H), W2, b2.reshape(1, H),
      W3, b3.reshape(1, H), g.reshape(1, H), beta.reshape(1, H))


def _sc_gather(u, v, src, dst, chunk=200):
    E = src.shape[0]
    rows_per_w = E // _NW
    n_chunks = rows_per_w // chunk
    mesh = plsc.VectorSubcoreMesh(core_axis_name="c", subcore_axis_name="s")

    @functools.partial(
        pl.kernel,
        out_type=jax.ShapeDtypeStruct((E, H), jnp.float32),
        mesh=mesh,
        scratch_types=[
            pltpu.VMEM((rows_per_w,), jnp.int32),
            pltpu.VMEM((rows_per_w,), jnp.int32),
            pltpu.VMEM((chunk, H), jnp.float32),
            pltpu.VMEM((chunk, H), jnp.float32),
            pltpu.VMEM((chunk, H), jnp.float32),
            pltpu.VMEM((chunk, H), jnp.float32),
            pltpu.SemaphoreType.DMA,
            pltpu.SemaphoreType.DMA,
            pltpu.SemaphoreType.DMA,
            pltpu.SemaphoreType.DMA,
        ],
    )
    def k(u_hbm, v_hbm, src_hbm, dst_hbm, out_hbm,
          isrc, idst, ru0, ru1, rv0, rv1, sg0, sg1, so0, so1):
        wid = lax.axis_index("s") * _NC + lax.axis_index("c")
        base0 = wid * rows_per_w
        ru = (ru0, ru1)
        rv = (rv0, rv1)
        sg = (sg0, sg1)
        so = (so0, so1)

        pltpu.sync_copy(src_hbm.at[pl.ds(base0, rows_per_w)], isrc)
        pltpu.sync_copy(dst_hbm.at[pl.ds(base0, rows_per_w)], idst)

        def issue(i, b):
            pltpu.async_copy(u_hbm.at[isrc.at[pl.ds(i * chunk, chunk)]],
                             ru[b], sg[b])
            pltpu.async_copy(v_hbm.at[idst.at[pl.ds(i * chunk, chunk)]],
                             rv[b], sg[b])

        def wait_gather(b):
            pltpu.make_async_copy(u_hbm.at[isrc.at[pl.ds(0, chunk)]],
                                  ru[b], sg[b]).wait()
            pltpu.make_async_copy(v_hbm.at[idst.at[pl.ds(0, chunk)]],
                                  rv[b], sg[b]).wait()

        def wait_out(b):
            pltpu.make_async_copy(ru[b], out_hbm.at[pl.ds(base0, chunk)],
                                  so[b]).wait()

        issue(0, 0)

        def outer(io, carry):
            for b in range(2):
                i = 2 * io + b
                nb = 1 - b

                @pl.when(i + 1 < n_chunks)
                def _issue_next():
                    @pl.when(i >= 1)
                    def _drain():
                        wait_out(nb)

                    issue(i + 1, nb)

                wait_gather(b)

                def add_row(r, c2):
                    for kk in range(H // 16):
                        sl = pl.ds(kk * 16, 16)
                        ru[b][r, sl] = ru[b][r, sl] + rv[b][r, sl]
                    return c2

                lax.fori_loop(0, chunk, add_row, 0)
                pltpu.async_copy(ru[b],
                                 out_hbm.at[pl.ds(base0 + i * chunk, chunk)],
                                 so[b])
            return carry

        lax.fori_loop(0, n_chunks // 2, outer, 0)
        wait_out(0)
        wait_out(1)

    return k(u, v, src, dst)


def _sc_scatter(e_new, dst, zeros, n_nodes, chunk=80):
    E = e_new.shape[0]
    rows_per_w = E // _NW
    n_chunks = rows_per_w // chunk
    rows_per_tile = 640
    n_pad = _NS * rows_per_tile
    last_rows = n_nodes - (_NS - 1) * rows_per_tile
    mesh = plsc.VectorSubcoreMesh(core_axis_name="c", subcore_axis_name="s")

    @functools.partial(
        pl.kernel,
        out_type=jax.ShapeDtypeStruct((2 * n_nodes, H), jnp.float32),
        mesh=mesh,
        scratch_types=[
            pltpu.VMEM((chunk,), jnp.int32),
            pltpu.VMEM((chunk,), jnp.int32),
            pltpu.VMEM((chunk, H), jnp.float32),
            pltpu.VMEM((chunk, H), jnp.float32),
            pltpu.VMEM_SHARED((n_pad, H), jnp.float32),
            pltpu.SemaphoreType.DMA,
            pltpu.SemaphoreType.DMA,
            pltpu.SemaphoreType.DMA,
            pltpu.SemaphoreType.DMA,
        ],
    )
    def k(enew_hbm, dst_hbm, zeros_hbm, out_hbm, idx0, idx1, buf0, buf1, acc,
          sr0, sr1, ss0, ss1):
        cid = lax.axis_index("c")
        sid = lax.axis_index("s")
        wid = sid * _NC + cid
        base0 = wid * rows_per_w
        nbase = sid * rows_per_tile
        idx = (idx0, idx1)
        buf = (buf0, buf1)
        sr = (sr0, sr1)
        ss = (ss0, ss1)
        pltpu.sync_copy(zeros_hbm, acc.at[pl.ds(nbase, rows_per_tile)])
        plsc.subcore_barrier()

        def issue(i, b):
            base = base0 + i * chunk
            pltpu.sync_copy(dst_hbm.at[pl.ds(base, chunk)], idx[b])
            pltpu.async_copy(enew_hbm.at[pl.ds(base, chunk)], buf[b], sr[b])

        def wait_rows(b):
            pltpu.make_async_copy(enew_hbm.at[pl.ds(base0, chunk)], buf[b],
                                  sr[b]).wait()

        issue(0, 0)

        def outer(io, carry):
            for b in range(2):
                i = 2 * io + b

                @pl.when(i + 1 < n_chunks)
                def _issue_next():
                    issue(i + 1, 1 - b)

                wait_rows(b)
                pltpu.sync_copy(buf[b], acc.at[idx[b]], add=True)
            return carry

        lax.fori_loop(0, n_chunks // 2, outer, 0)
        if n_chunks % 2 == 1:
            wait_rows(0)
            pltpu.sync_copy(buf[0], acc.at[idx[0]], add=True)
        plsc.subcore_barrier()

        @pl.when(sid < _NS - 1)
        def _full():
            pltpu.sync_copy(acc.at[pl.ds(nbase, rows_per_tile)],
                            out_hbm.at[pl.ds(cid * n_nodes + nbase,
                                             rows_per_tile)])

        @pl.when(sid == _NS - 1)
        def _tail():
            pltpu.sync_copy(acc.at[pl.ds((_NS - 1) * rows_per_tile, last_rows)],
                            out_hbm.at[pl.ds(cid * n_nodes
                                             + (_NS - 1) * rows_per_tile,
                                             last_rows)])

    return k(e_new, dst, zeros)


def kernel(x, edge_attr, edge_index, eb_W1, eb_b1, eb_W2, eb_b2, eb_W3, eb_b3,
           eb_g, eb_beta, nb_W1, nb_b1, nb_W2, nb_b2, nb_W3, nb_b3, nb_g, nb_beta):
    src = edge_index[0]
    dst = edge_index[1]
    u, v = _uv_precompute(x, eb_W1, eb_b1)
    ge = _sc_gather(u, v, src, dst)
    e_new, e_out = _edge_mlp(ge, edge_attr, eb_W1[2 * H:3 * H], eb_b1,
                             eb_W2, eb_b2, eb_W3, eb_b3, eb_g, eb_beta)
    zeros = jnp.zeros((640, H), jnp.float32)
    agg2 = _sc_scatter(e_new, dst, zeros, x.shape[0])
    x_out = _node_mlp(x, agg2, nb_W1, nb_b1, nb_W2, nb_b2, nb_W3, nb_b3,
                      nb_g, nb_beta)
    return (x_out, e_out)
